# baseline (speedup 1.0000x reference)
import jax
import jax.numpy as jnp
from jax.experimental import pallas as pl


def _add_body(x_ref, w_ref, o_ref):
    o_ref[...] = x_ref[...] + w_ref[...]


def kernel(x, pos_weight):
    B, L, H = x.shape
    Lb = 1024
    return pl.pallas_call(
        _add_body,
        grid=(L // Lb, B),
        in_specs=[
            pl.BlockSpec((None, Lb, H), lambda l, b: (b, l, 0)),
            pl.BlockSpec((Lb, H), lambda l, b: (l, 0)),
        ],
        out_specs=pl.BlockSpec((None, Lb, H), lambda l, b: (b, l, 0)),
        out_shape=jax.ShapeDtypeStruct(x.shape, x.dtype),
    )(x, pos_weight)


# speedup vs baseline: 1.0662x; 1.0662x over previous
import jax
import jax.numpy as jnp
from jax.experimental import pallas as pl


def _add_body(x_ref, w_ref, o_ref):
    o_ref[...] = x_ref[...] + w_ref[...]


def kernel(x, pos_weight):
    B, L, H = x.shape
    Lb = 2048
    return pl.pallas_call(
        _add_body,
        grid=(L // Lb, B),
        in_specs=[
            pl.BlockSpec((None, Lb, H), lambda l, b: (b, l, 0)),
            pl.BlockSpec((Lb, H), lambda l, b: (l, 0)),
        ],
        out_specs=pl.BlockSpec((None, Lb, H), lambda l, b: (b, l, 0)),
        out_shape=jax.ShapeDtypeStruct(x.shape, x.dtype),
    )(x, pos_weight)


# speedup vs baseline: 1.0687x; 1.0024x over previous
import jax
import jax.numpy as jnp
from jax.experimental import pallas as pl
from jax.experimental.pallas import tpu as pltpu


def _add_body(x_ref, w_ref, o_ref):
    o_ref[...] = x_ref[...] + w_ref[...]


def kernel(x, pos_weight):
    B, L, H = x.shape
    Lb = 1024
    return pl.pallas_call(
        _add_body,
        grid=(L // Lb,),
        in_specs=[
            pl.BlockSpec((B, Lb, H), lambda l: (0, l, 0)),
            pl.BlockSpec((Lb, H), lambda l: (l, 0)),
        ],
        out_specs=pl.BlockSpec((B, Lb, H), lambda l: (0, l, 0)),
        out_shape=jax.ShapeDtypeStruct(x.shape, x.dtype),
    )(x, pos_weight)
